# baseline (speedup 1.0000x reference)
import functools

import jax
import jax.numpy as jnp
from jax import lax
from jax.experimental import pallas as pl
from jax.experimental.pallas import tpu as pltpu
from jax.experimental.pallas import tpu_sc as plsc

CAP = 2000
D = 4096
B = 512
NOISE_SCALE = 0.15

NC = 2
NS = 16
NW = NC * NS
BPW = B // NW
RCH = 8
NCHUNK = BPW // RCH
SBLK = 2
RBLK = 256
NRB = B // RBLK


def _sc_body(buf_hbm, idx_hbm, tmp_hbm, *scratch):
    idx_v = scratch[0]
    rows = scratch[1:1 + NCHUNK]
    gsem = scratch[1 + NCHUNK:1 + 2 * NCHUNK]
    osem = scratch[1 + 2 * NCHUNK:1 + 3 * NCHUNK]

    cid = lax.axis_index("c")
    sid = lax.axis_index("s")
    wid = sid * NC + cid
    base = wid * BPW
    pltpu.sync_copy(idx_hbm.at[wid], idx_v)

    gh = [None] * NCHUNK
    for k in range(NCHUNK):
        gh[k] = pltpu.async_copy(buf_hbm.at[idx_v.at[k]], rows[k], gsem[k])
    oh = [None] * NCHUNK
    for k in range(NCHUNK):
        gh[k].wait()
        oh[k] = pltpu.async_copy(
            rows[k], tmp_hbm.at[pl.ds(base + k * RCH, RCH)], osem[k])
    for k in range(NCHUNK):
        oh[k].wait()


def _stats_body(buf_ref, mean_ref, rstd_ref, s1_ref, s2_ref):
    i = pl.program_id(0)
    x = buf_ref[...]
    ps1 = jnp.sum(x, axis=0)
    ps2 = jnp.sum(x * x, axis=0)

    @pl.when(i == 0)
    def _():
        s1_ref[...] = ps1
        s2_ref[...] = ps2

    @pl.when(i > 0)
    def _():
        s1_ref[...] += ps1
        s2_ref[...] += ps2

    @pl.when(i == SBLK - 1)
    def _():
        n = jnp.float32(CAP)
        s1 = s1_ref[...]
        s2 = s2_ref[...]
        m = s1 / n
        var = (s2 - s1 * s1 / n) / (n - 1.0)
        std = jnp.maximum(jnp.sqrt(var), 1e-6)
        mean_ref[...] = m
        rstd_ref[...] = 1.0 / std


def _norm_body(tmp_ref, noise_ref, mean_ref, rstd_ref, out_ref):
    out_ref[...] = (
        tmp_ref[...] + noise_ref[...] * NOISE_SCALE - mean_ref[...][None, :]
    ) * rstd_ref[...][None, :]


def kernel(buffer, indices, noise):
    idx3 = jnp.reshape(indices, (NW, NCHUNK, RCH))

    mesh = plsc.VectorSubcoreMesh(core_axis_name="c", subcore_axis_name="s")
    scratch = [pltpu.VMEM((NCHUNK, RCH), jnp.int32)]
    scratch += [pltpu.VMEM((RCH, D), jnp.float32) for _ in range(NCHUNK)]
    scratch += [pltpu.SemaphoreType.DMA for _ in range(2 * NCHUNK)]
    tmp = pl.kernel(
        _sc_body,
        mesh=mesh,
        out_type=jax.ShapeDtypeStruct((B, D), jnp.float32),
        scratch_types=scratch,
    )(buffer, idx3)

    mean, rstd = pl.pallas_call(
        _stats_body,
        grid=(SBLK,),
        in_specs=[pl.BlockSpec((CAP // SBLK, D), lambda i: (i, 0))],
        out_specs=[pl.BlockSpec((D,), lambda i: (0,)),
                   pl.BlockSpec((D,), lambda i: (0,))],
        out_shape=[jax.ShapeDtypeStruct((D,), jnp.float32),
                   jax.ShapeDtypeStruct((D,), jnp.float32)],
        scratch_shapes=[pltpu.VMEM((D,), jnp.float32),
                        pltpu.VMEM((D,), jnp.float32)],
    )(buffer)

    return pl.pallas_call(
        _norm_body,
        grid=(NRB,),
        in_specs=[pl.BlockSpec((RBLK, D), lambda i: (i, 0)),
                  pl.BlockSpec((RBLK, D), lambda i: (i, 0)),
                  pl.BlockSpec((D,), lambda i: (0,)),
                  pl.BlockSpec((D,), lambda i: (0,))],
        out_specs=pl.BlockSpec((RBLK, D), lambda i: (i, 0)),
        out_shape=jax.ShapeDtypeStruct((B, D), jnp.float32),
    )(tmp, noise, mean, rstd)


# speedup vs baseline: 1.0230x; 1.0230x over previous
import functools

import jax
import jax.numpy as jnp
from jax import lax
from jax.experimental import pallas as pl
from jax.experimental.pallas import tpu as pltpu
from jax.experimental.pallas import tpu_sc as plsc

CAP = 2000
D = 4096
B = 512
NOISE_SCALE = 0.15

NC = 2
NS = 16
NW = NC * NS
BPW = B // NW
RCH = 4
NCHUNK = BPW // RCH
SBLK = 5
RBLK = 128
NRB = B // RBLK


def _sc_body(buf_hbm, idx_hbm, tmp_hbm, *scratch):
    idx_v = scratch[0]
    rows = scratch[1:1 + NCHUNK]
    gsem = scratch[1 + NCHUNK:1 + 2 * NCHUNK]
    osem = scratch[1 + 2 * NCHUNK:1 + 3 * NCHUNK]

    cid = lax.axis_index("c")
    sid = lax.axis_index("s")
    wid = sid * NC + cid
    base = wid * BPW
    pltpu.sync_copy(idx_hbm.at[wid], idx_v)

    gh = [None] * NCHUNK
    for k in range(NCHUNK):
        gh[k] = pltpu.async_copy(buf_hbm.at[idx_v.at[k]], rows[k], gsem[k])
    oh = [None] * NCHUNK
    for k in range(NCHUNK):
        gh[k].wait()
        oh[k] = pltpu.async_copy(
            rows[k], tmp_hbm.at[pl.ds(base + k * RCH, RCH)], osem[k])
    for k in range(NCHUNK):
        oh[k].wait()


def _stats_body(buf_ref, mean_ref, rstd_ref, s1_ref, s2_ref):
    i = pl.program_id(0)
    x = buf_ref[...]
    ps1 = jnp.sum(x, axis=0)
    ps2 = jnp.sum(x * x, axis=0)

    @pl.when(i == 0)
    def _():
        s1_ref[...] = ps1
        s2_ref[...] = ps2

    @pl.when(i > 0)
    def _():
        s1_ref[...] += ps1
        s2_ref[...] += ps2

    @pl.when(i == SBLK - 1)
    def _():
        n = jnp.float32(CAP)
        s1 = s1_ref[...]
        s2 = s2_ref[...]
        m = s1 / n
        var = (s2 - s1 * s1 / n) / (n - 1.0)
        std = jnp.maximum(jnp.sqrt(var), 1e-6)
        mean_ref[...] = m
        rstd_ref[...] = 1.0 / std


def _norm_body(tmp_ref, noise_ref, mean_ref, rstd_ref, out_ref):
    out_ref[...] = (
        tmp_ref[...] + noise_ref[...] * NOISE_SCALE - mean_ref[...][None, :]
    ) * rstd_ref[...][None, :]


def kernel(buffer, indices, noise):
    idx3 = jnp.reshape(indices, (NW, NCHUNK, RCH))

    mesh = plsc.VectorSubcoreMesh(core_axis_name="c", subcore_axis_name="s")
    scratch = [pltpu.VMEM((NCHUNK, RCH), jnp.int32)]
    scratch += [pltpu.VMEM((RCH, D), jnp.float32) for _ in range(NCHUNK)]
    scratch += [pltpu.SemaphoreType.DMA for _ in range(2 * NCHUNK)]
    tmp = pl.kernel(
        _sc_body,
        mesh=mesh,
        out_type=jax.ShapeDtypeStruct((B, D), jnp.float32),
        scratch_types=scratch,
    )(buffer, idx3)

    mean, rstd = pl.pallas_call(
        _stats_body,
        grid=(SBLK,),
        in_specs=[pl.BlockSpec((CAP // SBLK, D), lambda i: (i, 0))],
        out_specs=[pl.BlockSpec((D,), lambda i: (0,)),
                   pl.BlockSpec((D,), lambda i: (0,))],
        out_shape=[jax.ShapeDtypeStruct((D,), jnp.float32),
                   jax.ShapeDtypeStruct((D,), jnp.float32)],
        scratch_shapes=[pltpu.VMEM((D,), jnp.float32),
                        pltpu.VMEM((D,), jnp.float32)],
    )(buffer)

    return pl.pallas_call(
        _norm_body,
        grid=(NRB,),
        in_specs=[pl.BlockSpec((RBLK, D), lambda i: (i, 0)),
                  pl.BlockSpec((RBLK, D), lambda i: (i, 0)),
                  pl.BlockSpec((D,), lambda i: (0,)),
                  pl.BlockSpec((D,), lambda i: (0,))],
        out_specs=pl.BlockSpec((RBLK, D), lambda i: (i, 0)),
        out_shape=jax.ShapeDtypeStruct((B, D), jnp.float32),
    )(tmp, noise, mean, rstd)


# speedup vs baseline: 1.0268x; 1.0037x over previous
import functools

import jax
import jax.numpy as jnp
from jax import lax
from jax.experimental import pallas as pl
from jax.experimental.pallas import tpu as pltpu
from jax.experimental.pallas import tpu_sc as plsc

CAP = 2000
D = 4096
B = 512
NOISE_SCALE = 0.15

NC = 2
NS = 16
NW = NC * NS
BPW = B // NW
RCH = 4
NCHUNK = BPW // RCH
SBLK = 5
RBLK = 256
NRB = B // RBLK


def _sc_body(buf_hbm, idx_hbm, tmp_hbm, *scratch):
    idx_v = scratch[0]
    rows = scratch[1:1 + NCHUNK]
    gsem = scratch[1 + NCHUNK:1 + 2 * NCHUNK]
    osem = scratch[1 + 2 * NCHUNK:1 + 3 * NCHUNK]

    cid = lax.axis_index("c")
    sid = lax.axis_index("s")
    wid = sid * NC + cid
    base = wid * BPW
    pltpu.sync_copy(idx_hbm.at[wid], idx_v)

    gh = [None] * NCHUNK
    for k in range(NCHUNK):
        gh[k] = pltpu.async_copy(buf_hbm.at[idx_v.at[k]], rows[k], gsem[k])
    oh = [None] * NCHUNK
    for k in range(NCHUNK):
        gh[k].wait()
        oh[k] = pltpu.async_copy(
            rows[k], tmp_hbm.at[pl.ds(base + k * RCH, RCH)], osem[k])
    for k in range(NCHUNK):
        oh[k].wait()


def _stats_body(buf_ref, mean_ref, rstd_ref, s1_ref, s2_ref):
    i = pl.program_id(0)
    x = buf_ref[...]
    ps1 = jnp.sum(x, axis=0)
    ps2 = jnp.sum(x * x, axis=0)

    @pl.when(i == 0)
    def _():
        s1_ref[...] = ps1
        s2_ref[...] = ps2

    @pl.when(i > 0)
    def _():
        s1_ref[...] += ps1
        s2_ref[...] += ps2

    @pl.when(i == SBLK - 1)
    def _():
        n = jnp.float32(CAP)
        s1 = s1_ref[...]
        s2 = s2_ref[...]
        m = s1 / n
        var = (s2 - s1 * s1 / n) / (n - 1.0)
        std = jnp.maximum(jnp.sqrt(var), 1e-6)
        mean_ref[...] = m
        rstd_ref[...] = 1.0 / std


def _norm_body(tmp_ref, noise_ref, mean_ref, rstd_ref, out_ref):
    out_ref[...] = (
        tmp_ref[...] + noise_ref[...] * NOISE_SCALE - mean_ref[...][None, :]
    ) * rstd_ref[...][None, :]


def kernel(buffer, indices, noise):
    idx3 = jnp.reshape(indices, (NW, NCHUNK, RCH))

    mesh = plsc.VectorSubcoreMesh(core_axis_name="c", subcore_axis_name="s")
    scratch = [pltpu.VMEM((NCHUNK, RCH), jnp.int32)]
    scratch += [pltpu.VMEM((RCH, D), jnp.float32) for _ in range(NCHUNK)]
    scratch += [pltpu.SemaphoreType.DMA for _ in range(2 * NCHUNK)]
    tmp = pl.kernel(
        _sc_body,
        mesh=mesh,
        out_type=jax.ShapeDtypeStruct((B, D), jnp.float32),
        scratch_types=scratch,
    )(buffer, idx3)

    mean, rstd = pl.pallas_call(
        _stats_body,
        grid=(SBLK,),
        in_specs=[pl.BlockSpec((CAP // SBLK, D), lambda i: (i, 0))],
        out_specs=[pl.BlockSpec((D,), lambda i: (0,)),
                   pl.BlockSpec((D,), lambda i: (0,))],
        out_shape=[jax.ShapeDtypeStruct((D,), jnp.float32),
                   jax.ShapeDtypeStruct((D,), jnp.float32)],
        scratch_shapes=[pltpu.VMEM((D,), jnp.float32),
                        pltpu.VMEM((D,), jnp.float32)],
    )(buffer)

    return pl.pallas_call(
        _norm_body,
        grid=(NRB,),
        in_specs=[pl.BlockSpec((RBLK, D), lambda i: (i, 0)),
                  pl.BlockSpec((RBLK, D), lambda i: (i, 0)),
                  pl.BlockSpec((D,), lambda i: (0,)),
                  pl.BlockSpec((D,), lambda i: (0,))],
        out_specs=pl.BlockSpec((RBLK, D), lambda i: (i, 0)),
        out_shape=jax.ShapeDtypeStruct((B, D), jnp.float32),
    )(tmp, noise, mean, rstd)


# speedup vs baseline: 1.0286x; 1.0018x over previous
import functools

import jax
import jax.numpy as jnp
from jax import lax
from jax.experimental import pallas as pl
from jax.experimental.pallas import tpu as pltpu
from jax.experimental.pallas import tpu_sc as plsc

CAP = 2000
D = 4096
B = 512
NOISE_SCALE = 0.15

NC = 2
NS = 16
NW = NC * NS
BPW = B // NW
RCH = 4
NCHUNK = BPW // RCH
SBLK = 5
RBLK = 256
NRB = B // RBLK


def _sc_body(buf_hbm, idx_hbm, tmp_hbm, *scratch):
    idx_v = scratch[0]
    rows = scratch[1:1 + NCHUNK]
    gsem = scratch[1 + NCHUNK:1 + 2 * NCHUNK]
    osem = scratch[1 + 2 * NCHUNK:1 + 3 * NCHUNK]

    cid = lax.axis_index("c")
    sid = lax.axis_index("s")
    wid = sid * NC + cid
    base = wid * BPW
    pltpu.sync_copy(idx_hbm.at[wid], idx_v)

    gh = [None] * NCHUNK
    for k in range(NCHUNK):
        gh[k] = pltpu.async_copy(buf_hbm.at[idx_v.at[k]], rows[k], gsem[k])
    oh = [None] * NCHUNK
    for k in range(NCHUNK):
        gh[k].wait()
        oh[k] = pltpu.async_copy(
            rows[k], tmp_hbm.at[pl.ds(base + k * RCH, RCH)], osem[k])
    for k in range(NCHUNK):
        oh[k].wait()


def _stats_body(buf_ref, mean_ref, rstd_ref, s1_ref, s2_ref):
    i = pl.program_id(0)
    x = buf_ref[...]
    ones8 = jnp.ones((8, CAP // SBLK), jnp.float32)
    dn = (((1,), (0,)), ((), ()))
    ps1 = lax.dot_general(ones8, x, dn, preferred_element_type=jnp.float32)
    ps2 = lax.dot_general(ones8, x * x, dn, preferred_element_type=jnp.float32)

    @pl.when(i == 0)
    def _():
        s1_ref[...] = ps1
        s2_ref[...] = ps2

    @pl.when(i > 0)
    def _():
        s1_ref[...] += ps1
        s2_ref[...] += ps2

    @pl.when(i == SBLK - 1)
    def _():
        n = jnp.float32(CAP)
        s1 = s1_ref[0]
        s2 = s2_ref[0]
        m = s1 / n
        var = (s2 - s1 * s1 / n) / (n - 1.0)
        std = jnp.maximum(jnp.sqrt(var), 1e-6)
        mean_ref[...] = m
        rstd_ref[...] = 1.0 / std


def _norm_body(tmp_ref, noise_ref, mean_ref, rstd_ref, out_ref):
    out_ref[...] = (
        tmp_ref[...] + noise_ref[...] * NOISE_SCALE - mean_ref[...][None, :]
    ) * rstd_ref[...][None, :]


def kernel(buffer, indices, noise):
    idx3 = jnp.reshape(indices, (NW, NCHUNK, RCH))

    mesh = plsc.VectorSubcoreMesh(core_axis_name="c", subcore_axis_name="s")
    scratch = [pltpu.VMEM((NCHUNK, RCH), jnp.int32)]
    scratch += [pltpu.VMEM((RCH, D), jnp.float32) for _ in range(NCHUNK)]
    scratch += [pltpu.SemaphoreType.DMA for _ in range(2 * NCHUNK)]
    tmp = pl.kernel(
        _sc_body,
        mesh=mesh,
        out_type=jax.ShapeDtypeStruct((B, D), jnp.float32),
        scratch_types=scratch,
    )(buffer, idx3)

    mean, rstd = pl.pallas_call(
        _stats_body,
        grid=(SBLK,),
        in_specs=[pl.BlockSpec((CAP // SBLK, D), lambda i: (i, 0))],
        out_specs=[pl.BlockSpec((D,), lambda i: (0,)),
                   pl.BlockSpec((D,), lambda i: (0,))],
        out_shape=[jax.ShapeDtypeStruct((D,), jnp.float32),
                   jax.ShapeDtypeStruct((D,), jnp.float32)],
        scratch_shapes=[pltpu.VMEM((8, D), jnp.float32),
                        pltpu.VMEM((8, D), jnp.float32)],
    )(buffer)

    return pl.pallas_call(
        _norm_body,
        grid=(NRB,),
        in_specs=[pl.BlockSpec((RBLK, D), lambda i: (i, 0)),
                  pl.BlockSpec((RBLK, D), lambda i: (i, 0)),
                  pl.BlockSpec((D,), lambda i: (0,)),
                  pl.BlockSpec((D,), lambda i: (0,))],
        out_specs=pl.BlockSpec((RBLK, D), lambda i: (i, 0)),
        out_shape=jax.ShapeDtypeStruct((B, D), jnp.float32),
    )(tmp, noise, mean, rstd)
